# baseline (speedup 1.0000x reference)
import jax
import jax.numpy as jnp
from jax.experimental import pallas as pl

NUM_EXPERTS = 3
IN_FEATURES = 1024
OUT_FEATURES = 1024
N_TOKENS = 8192
TOKEN_BLOCK = 1024


def _body(x_ref, ids_ref, w_ref, out_ref):
    e = pl.program_id(1)
    x = x_ref[...]
    ids = ids_ref[...]
    y = jax.lax.dot_general(
        x, w_ref[0],
        dimension_numbers=(((1,), (1,)), ((), ())),
        preferred_element_type=jnp.float32,
    )
    yv = jnp.where(ids == e.astype(jnp.float32), y, 0.0)

    @pl.when(e == 0)
    def _init():
        out_ref[...] = yv

    @pl.when(e > 0)
    def _accum():
        out_ref[...] += yv


def kernel(x, modality_ids, weight):
    w = weight.reshape(NUM_EXPERTS, OUT_FEATURES, IN_FEATURES)
    ids_f = modality_ids.astype(jnp.float32).reshape(N_TOKENS, 1)
    nb = N_TOKENS // TOKEN_BLOCK
    return pl.pallas_call(
        _body,
        grid=(nb, NUM_EXPERTS),
        in_specs=[
            pl.BlockSpec((TOKEN_BLOCK, IN_FEATURES), lambda i, e: (i, 0)),
            pl.BlockSpec((TOKEN_BLOCK, 1), lambda i, e: (i, 0)),
            pl.BlockSpec((1, OUT_FEATURES, IN_FEATURES), lambda i, e: (e, 0, 0)),
        ],
        out_specs=pl.BlockSpec((TOKEN_BLOCK, OUT_FEATURES), lambda i, e: (i, 0)),
        out_shape=jax.ShapeDtypeStruct((N_TOKENS, OUT_FEATURES), jnp.float32),
    )(x, ids_f, w)
